# baseline (speedup 1.0000x reference)
import jax
import jax.numpy as jnp
from jax import lax
from jax.experimental import pallas as pl
from jax.experimental.pallas import tpu as pltpu

_C = 18
_K = 5
_L = 320
_GRP = 32
_OUTP = 8
_SLOPE = 0.01
_EPS = 1e-5
_PT = 40
_HALO = 8


def _conv_kernel(xl_ref, xc_ref, xr_ref, w1s_ref, t1_ref, w2s_ref, t2_ref,
                 out_ref):
    i = pl.program_id(0)
    nlast = pl.num_programs(0) - 1
    nb = xc_ref.shape[2]
    zf32 = jnp.zeros((), jnp.float32)

    xl = (jnp.where(i == 0, zf32, 1.0)
          * xl_ref[...].reshape(_C, _HALO * nb)[:, 4 * nb:])
    xr = (jnp.where(i == nlast, zf32, 1.0)
          * xr_ref[...].reshape(_C, _HALO * nb)[:, :4 * nb])
    xc = xc_ref[...].reshape(_C, _PT * nb)
    ext = jnp.concatenate([xl, xc, xr],
                          axis=1).astype(jnp.bfloat16)

    ne1 = (_PT + 4) * nb
    zpad = jnp.zeros((_GRP - _C, ne1), jnp.bfloat16)
    groups = []
    for k in range(_K):
        groups.append(ext[:, k * nb:k * nb + ne1])
        groups.append(zpad)
    xs = jnp.concatenate(groups, axis=0)

    w1 = w1s_ref[...]
    wr = w2s_ref[:, 120:120 + _C]
    zrow = jnp.zeros((1, _C), jnp.float32)
    wcols = []
    for k in range(_K):
        blk = jnp.concatenate(
            [w1[:, 24 * k:24 * k + _C], wr if k == 2 else zrow], axis=0)
        wcols.append(jnp.pad(blk, ((0, 0), (0, _GRP - _C))))
    w1c = jnp.concatenate(wcols, axis=1).astype(jnp.bfloat16)

    y = jnp.dot(w1c, xs, preferred_element_type=jnp.float32)
    h = y[:_C, :] + t1_ref[...]
    h = jnp.maximum(h, _SLOPE * h)
    lane = lax.broadcasted_iota(jnp.int32, (1, ne1), 1)
    bad = ((i == 0) & (lane < 2 * nb)) | ((i == nlast) & (lane >= ne1 - 2 * nb))
    h = jnp.where(bad, zf32, h)

    w2z = jnp.concatenate(
        [w2s_ref[:, 24 * k:24 * k + _C] for k in range(_K)],
        axis=0).astype(jnp.bfloat16)
    z = jnp.dot(w2z, h.astype(jnp.bfloat16),
                preferred_element_type=jnp.float32)
    no = _PT * nb
    o = y[_C:_C + 1, 2 * nb:2 * nb + no] + t2_ref[...]
    for k in range(_K):
        o = o + z[k:k + 1, k * nb:k * nb + no]
    o = jnp.maximum(o, _SLOPE * o)
    out_ref[...] = o.reshape(_PT, nb).astype(jnp.bfloat16)


def _layer_norm(x, g, b):
    mu = jnp.mean(x, axis=-1, keepdims=True)
    m2 = jnp.mean(x * x, axis=-1, keepdims=True)
    s = lax.rsqrt(m2 - mu * mu + _EPS)
    return x * (s * g) + (b - mu * s * g)


def _head_kernel(xc_ref, pw_ref, bp_ref, wvo_ref, bvo_ref, l1g_ref, l1b_ref,
                 w1_ref, b1_ref, w2_ref, b2_ref, l2g_ref, l2b_ref,
                 wd_ref, bd_ref, wf_ref, bf_ref, out_ref):
    pwb = pw_ref[2:2 + _L, :].astype(jnp.bfloat16)
    x = jnp.dot(xc_ref[...], pwb,
                preferred_element_type=jnp.float32) + bp_ref[...]
    for l in range(4):
        attn = jnp.dot(x.astype(jnp.bfloat16), wvo_ref[l].astype(jnp.bfloat16),
                       preferred_element_type=jnp.float32) + bvo_ref[l]
        x = _layer_norm(x + attn, l1g_ref[l], l1b_ref[l])
        ff = jnp.dot(x.astype(jnp.bfloat16), w1_ref[l].astype(jnp.bfloat16),
                     preferred_element_type=jnp.float32) + b1_ref[l]
        ff = jnp.maximum(ff, 0.0).astype(jnp.bfloat16)
        ff = jnp.dot(ff, w2_ref[l].astype(jnp.bfloat16),
                     preferred_element_type=jnp.float32) + b2_ref[l]
        s = x + ff
        if l < 3:
            x = _layer_norm(s, l2g_ref[l], l2b_ref[l])
        else:
            mu = jnp.mean(s, axis=-1, keepdims=True)
            m2 = jnp.mean(s * s, axis=-1, keepdims=True)
            r = lax.rsqrt(m2 - mu * mu + _EPS)
            x = (s - mu) * r * l2g_ref[3]
    wfb = wf_ref[:, :_OUTP].astype(jnp.bfloat16)
    wdwf = jnp.dot(wd_ref[...].astype(jnp.bfloat16), wfb,
                   preferred_element_type=jnp.float32)
    wdwfb = wdwf.astype(jnp.bfloat16)
    bdf = (jnp.dot(l2b_ref[3].astype(jnp.bfloat16), wdwfb,
                   preferred_element_type=jnp.float32)
           + jnp.dot(bd_ref[...].astype(jnp.bfloat16), wfb,
                     preferred_element_type=jnp.float32)
           + bf_ref[:, :_OUTP])
    out_ref[...] = jnp.dot(x.astype(jnp.bfloat16), wdwfb,
                           preferred_element_type=jnp.float32) + bdf


def _full(a):
    nd = a.ndim
    return pl.BlockSpec(a.shape, lambda i, nd=nd: (0,) * nd)


def kernel(x, w1s, t1, w2s, t2, PW, bp, Wvo, bvo, ln1g, ln1b,
           W1, b1, W2, b2, ln2g, ln2b, Wd, bd, Wfp, bfp):
    B = x.shape[0]
    Bp = -(-max(B, 1) // 8) * 8
    if Bp != B:
        x = jnp.pad(x, ((0, Bp - B), (0, 0), (0, 0)))

    bth = min(1024, Bp)
    while Bp % bth:
        bth -= 8

    xf = jnp.transpose(x, (1, 2, 0))

    cparams = pltpu.CompilerParams(dimension_semantics=("parallel",),
                                   vmem_limit_bytes=100 * 1024 * 1024)

    nt = 320 // _PT
    conv_in = [xf, xf, xf, w1s, t1, w2s, t2]
    conv_specs = (
        [pl.BlockSpec((_C, _HALO, Bp),
                      lambda i: (0, jnp.maximum(i * (_PT // _HALO) - 1, 0), 0)),
         pl.BlockSpec((_C, _PT, Bp), lambda i: (0, i, 0)),
         pl.BlockSpec((_C, _HALO, Bp),
                      lambda i: (0, jnp.minimum((i + 1) * (_PT // _HALO),
                                                _L // _HALO - 1), 0))]
        + [_full(a) for a in conv_in[3:]])
    convout = pl.pallas_call(
        _conv_kernel,
        out_shape=jax.ShapeDtypeStruct((_L, Bp), jnp.bfloat16),
        grid=(nt,),
        in_specs=conv_specs,
        out_specs=pl.BlockSpec((_PT, Bp), lambda i: (i, 0)),
        compiler_params=cparams,
    )(*conv_in)

    convT = convout.T

    head_in = [convT, PW, bp, Wvo, bvo, ln1g, ln1b,
               W1, b1, W2, b2, ln2g, ln2b, Wd, bd, Wfp, bfp]
    head_specs = ([pl.BlockSpec((bth, _L), lambda i: (i, 0))]
                  + [_full(a) for a in head_in[1:]])
    logits = pl.pallas_call(
        _head_kernel,
        out_shape=jax.ShapeDtypeStruct((Bp, _OUTP), jnp.float32),
        grid=(Bp // bth,),
        in_specs=head_specs,
        out_specs=pl.BlockSpec((bth, _OUTP), lambda i: (i, 0)),
        compiler_params=cparams,
    )(*head_in)

    return logits[:B, :2].reshape(B, 1, 2)


# speedup vs baseline: 1.0061x; 1.0061x over previous
import jax
import jax.numpy as jnp
from jax import lax
from jax.experimental import pallas as pl
from jax.experimental.pallas import tpu as pltpu

_C = 18
_K = 5
_L = 320
_GRP = 32
_OUTP = 128
_SLOPE = 0.01
_EPS = 1e-5
_PT = 40
_HALO = 8


def _conv_kernel(xl_ref, xc_ref, xr_ref, w1s_ref, t1_ref, w2s_ref, t2_ref,
                 out_ref):
    i = pl.program_id(0)
    nlast = pl.num_programs(0) - 1
    nb = xc_ref.shape[2]
    zf32 = jnp.zeros((), jnp.float32)

    xl = (jnp.where(i == 0, zf32, 1.0)
          * xl_ref[...].reshape(_C, _HALO * nb)[:, 4 * nb:])
    xr = (jnp.where(i == nlast, zf32, 1.0)
          * xr_ref[...].reshape(_C, _HALO * nb)[:, :4 * nb])
    xc = xc_ref[...].reshape(_C, _PT * nb)
    ext = jnp.concatenate([xl, xc, xr],
                          axis=1).astype(jnp.bfloat16)

    ne1 = (_PT + 4) * nb
    zpad = jnp.zeros((_GRP - _C, ne1), jnp.bfloat16)
    groups = []
    for k in range(_K):
        groups.append(ext[:, k * nb:k * nb + ne1])
        groups.append(zpad)
    xs = jnp.concatenate(groups, axis=0)

    w1 = w1s_ref[...]
    wr = w2s_ref[:, 120:120 + _C]
    zrow = jnp.zeros((1, _C), jnp.float32)
    wcols = []
    for k in range(_K):
        blk = jnp.concatenate(
            [w1[:, 24 * k:24 * k + _C], wr if k == 2 else zrow], axis=0)
        wcols.append(jnp.pad(blk, ((0, 0), (0, _GRP - _C))))
    w1c = jnp.concatenate(wcols, axis=1).astype(jnp.bfloat16)

    y = jnp.dot(w1c, xs, preferred_element_type=jnp.float32)
    h = y[:_C, :] + t1_ref[...]
    h = jnp.maximum(h, _SLOPE * h)
    lane = lax.broadcasted_iota(jnp.int32, (1, ne1), 1)
    bad = ((i == 0) & (lane < 2 * nb)) | ((i == nlast) & (lane >= ne1 - 2 * nb))
    h = jnp.where(bad, zf32, h)

    w2z = jnp.concatenate(
        [w2s_ref[:, 24 * k:24 * k + _C] for k in range(_K)],
        axis=0).astype(jnp.bfloat16)
    z = jnp.dot(w2z, h.astype(jnp.bfloat16),
                preferred_element_type=jnp.float32)
    no = _PT * nb
    o = y[_C:_C + 1, 2 * nb:2 * nb + no] + t2_ref[...]
    for k in range(_K):
        o = o + z[k:k + 1, k * nb:k * nb + no]
    o = jnp.maximum(o, _SLOPE * o)
    out_ref[...] = o.reshape(_PT, nb).astype(jnp.bfloat16)


def _layer_norm(x, g, b):
    mu = jnp.mean(x, axis=-1, keepdims=True)
    xc = x - mu
    var = jnp.mean(xc * xc, axis=-1, keepdims=True)
    return xc * lax.rsqrt(var + _EPS) * g + b


def _head_kernel(xc_ref, pw_ref, bp_ref, wvo_ref, bvo_ref, l1g_ref, l1b_ref,
                 w1_ref, b1_ref, w2_ref, b2_ref, l2g_ref, l2b_ref,
                 wd_ref, bd_ref, wf_ref, bf_ref, out_ref):
    pwb = pw_ref[2:2 + _L, :].astype(jnp.bfloat16)
    x = jnp.dot(xc_ref[...], pwb,
                preferred_element_type=jnp.float32) + bp_ref[...]
    for l in range(4):
        attn = jnp.dot(x.astype(jnp.bfloat16), wvo_ref[l].astype(jnp.bfloat16),
                       preferred_element_type=jnp.float32) + bvo_ref[l]
        x = _layer_norm(x + attn, l1g_ref[l], l1b_ref[l])
        ff = jnp.dot(x.astype(jnp.bfloat16), w1_ref[l].astype(jnp.bfloat16),
                     preferred_element_type=jnp.float32) + b1_ref[l]
        ff = jnp.maximum(ff, 0.0).astype(jnp.bfloat16)
        ff = jnp.dot(ff, w2_ref[l].astype(jnp.bfloat16),
                     preferred_element_type=jnp.float32) + b2_ref[l]
        s = x + ff
        if l < 3:
            x = _layer_norm(s, l2g_ref[l], l2b_ref[l])
        else:
            mu = jnp.mean(s, axis=-1, keepdims=True)
            sc = s - mu
            var = jnp.mean(sc * sc, axis=-1, keepdims=True)
            x = sc * lax.rsqrt(var + _EPS) * l2g_ref[3]
    wfb = wf_ref[...].astype(jnp.bfloat16)
    wdwf = jnp.dot(wd_ref[...].astype(jnp.bfloat16), wfb,
                   preferred_element_type=jnp.float32)
    wdwfb = wdwf.astype(jnp.bfloat16)
    bdf = (jnp.dot(l2b_ref[3].astype(jnp.bfloat16), wdwfb,
                   preferred_element_type=jnp.float32)
           + jnp.dot(bd_ref[...].astype(jnp.bfloat16), wfb,
                     preferred_element_type=jnp.float32) + bf_ref[...])
    out_ref[...] = jnp.dot(x.astype(jnp.bfloat16), wdwfb,
                           preferred_element_type=jnp.float32) + bdf


def _full(a):
    nd = a.ndim
    return pl.BlockSpec(a.shape, lambda i, nd=nd: (0,) * nd)


def kernel(x, w1s, t1, w2s, t2, PW, bp, Wvo, bvo, ln1g, ln1b,
           W1, b1, W2, b2, ln2g, ln2b, Wd, bd, Wfp, bfp):
    B = x.shape[0]
    Bp = -(-max(B, 1) // 8) * 8
    if Bp != B:
        x = jnp.pad(x, ((0, Bp - B), (0, 0), (0, 0)))

    bth = min(1024, Bp)
    while Bp % bth:
        bth -= 8

    xf = jnp.transpose(x, (1, 2, 0))

    cparams = pltpu.CompilerParams(dimension_semantics=("parallel",),
                                   vmem_limit_bytes=100 * 1024 * 1024)

    nt = 320 // _PT
    conv_in = [xf, xf, xf, w1s, t1, w2s, t2]
    conv_specs = (
        [pl.BlockSpec((_C, _HALO, Bp),
                      lambda i: (0, jnp.maximum(i * (_PT // _HALO) - 1, 0), 0)),
         pl.BlockSpec((_C, _PT, Bp), lambda i: (0, i, 0)),
         pl.BlockSpec((_C, _HALO, Bp),
                      lambda i: (0, jnp.minimum((i + 1) * (_PT // _HALO),
                                                _L // _HALO - 1), 0))]
        + [_full(a) for a in conv_in[3:]])
    convout = pl.pallas_call(
        _conv_kernel,
        out_shape=jax.ShapeDtypeStruct((_L, Bp), jnp.bfloat16),
        grid=(nt,),
        in_specs=conv_specs,
        out_specs=pl.BlockSpec((_PT, Bp), lambda i: (i, 0)),
        compiler_params=cparams,
    )(*conv_in)

    convT = convout.T

    head_in = [convT, PW, bp, Wvo, bvo, ln1g, ln1b,
               W1, b1, W2, b2, ln2g, ln2b, Wd, bd, Wfp, bfp]
    head_specs = ([pl.BlockSpec((bth, _L), lambda i: (i, 0))]
                  + [_full(a) for a in head_in[1:]])
    logits = pl.pallas_call(
        _head_kernel,
        out_shape=jax.ShapeDtypeStruct((Bp, _OUTP), jnp.float32),
        grid=(Bp // bth,),
        in_specs=head_specs,
        out_specs=pl.BlockSpec((bth, _OUTP), lambda i: (i, 0)),
        compiler_params=cparams,
    )(*head_in)

    return logits[:B, :2].reshape(B, 1, 2)
